# baseline (speedup 1.0000x reference)
import functools

import jax
import jax.numpy as jnp
from jax.experimental import pallas as pl
from jax.experimental.pallas import tpu as pltpu

F_IN = 128
FILTERS = 128
BLOCK = 2560


def _fused_gru_head(x_ref, wz_ref, wh_ref, bz_ref, bh_ref, wlin_ref,
                    blin_ref, out_ref, w_scr):
    @pl.when(pl.program_id(0) == 0)
    def _fold_weights():
        w_scr[:, :FILTERS] = wz_ref[0, 0, :F_IN, :] + wz_ref[1, 0, :F_IN, :]
        w_scr[:, FILTERS:] = wh_ref[0, 0, :F_IN, :] + wh_ref[1, 0, :F_IN, :]

    y = jnp.dot(x_ref[...], w_scr[...], preferred_element_type=jnp.float32)
    z = jax.nn.sigmoid(y[:, :FILTERS] + bz_ref[...].reshape(1, FILTERS))
    h_tilde = jnp.tanh(y[:, FILTERS:] + bh_ref[...].reshape(1, FILTERS))
    h = jnp.maximum((1.0 - z) * h_tilde, 0.0)
    out_ref[...] = (
        jnp.sum(h * wlin_ref[...], axis=1, keepdims=True) + blin_ref[0]
    )


@functools.partial(jax.jit, static_argnames=())
def kernel(x, edge_index, edge_weight, W_z, b_z, W_r, b_r, W_h, b_h,
           W_lin, b_lin):
    del edge_index, edge_weight, W_r, b_r
    n = x.shape[0]
    cat = W_z.shape[2]

    grid = (pl.cdiv(n, BLOCK),)
    out = pl.pallas_call(
        _fused_gru_head,
        grid=grid,
        in_specs=[
            pl.BlockSpec((BLOCK, F_IN), lambda i: (i, 0)),
            pl.BlockSpec((2, 1, cat, FILTERS), lambda i: (0, 0, 0, 0)),
            pl.BlockSpec((2, 1, cat, FILTERS), lambda i: (0, 0, 0, 0)),
            pl.BlockSpec((FILTERS,), lambda i: (0,)),
            pl.BlockSpec((FILTERS,), lambda i: (0,)),
            pl.BlockSpec((1, FILTERS), lambda i: (0, 0)),
            pl.BlockSpec((1,), lambda i: (0,)),
        ],
        out_specs=pl.BlockSpec((BLOCK, 1), lambda i: (i, 0)),
        out_shape=jax.ShapeDtypeStruct((n, 1), x.dtype),
        scratch_shapes=[pltpu.VMEM((F_IN, 2 * FILTERS), jnp.float32)],
        compiler_params=pltpu.CompilerParams(
            dimension_semantics=("arbitrary",),
        ),
    )(x, W_z, W_h, b_z, b_h, W_lin, b_lin)
    return out


# speedup vs baseline: 1.1054x; 1.1054x over previous
import functools

import jax
import jax.numpy as jnp
from jax.experimental import pallas as pl
from jax.experimental.pallas import tpu as pltpu

F_IN = 128
FILTERS = 128
BLOCK = 5120


def _fused_gru_head(x_ref, wz_ref, wh_ref, bz_ref, bh_ref, wlin_ref,
                    blin_ref, out_ref, w_scr):
    @pl.when(pl.program_id(0) == 0)
    def _fold_weights():
        w_scr[:, :FILTERS] = wz_ref[0, 0] + wz_ref[1, 0]
        w_scr[:, FILTERS:] = wh_ref[0, 0] + wh_ref[1, 0]

    y = jnp.dot(x_ref[...], w_scr[...], preferred_element_type=jnp.float32)
    z = jax.nn.sigmoid(y[:, :FILTERS] + bz_ref[...].reshape(1, FILTERS))
    h_tilde = jnp.tanh(y[:, FILTERS:] + bh_ref[...].reshape(1, FILTERS))
    h = jnp.maximum((1.0 - z) * h_tilde, 0.0)
    out_ref[...] = (
        jnp.sum(h * wlin_ref[...], axis=1, keepdims=True) + blin_ref[0]
    )


@functools.partial(jax.jit, static_argnames=())
def kernel(x, edge_index, edge_weight, W_z, b_z, W_r, b_r, W_h, b_h,
           W_lin, b_lin):
    del edge_index, edge_weight, W_r, b_r
    n = x.shape[0]
    cat = W_z.shape[2]

    grid = (pl.cdiv(n, BLOCK),)
    out = pl.pallas_call(
        _fused_gru_head,
        grid=grid,
        in_specs=[
            pl.BlockSpec((BLOCK, F_IN), lambda i: (i, 0)),
            pl.BlockSpec((2, 1, F_IN, FILTERS), lambda i: (0, 0, 0, 0)),
            pl.BlockSpec((2, 1, F_IN, FILTERS), lambda i: (0, 0, 0, 0)),
            pl.BlockSpec((FILTERS,), lambda i: (0,)),
            pl.BlockSpec((FILTERS,), lambda i: (0,)),
            pl.BlockSpec((1, FILTERS), lambda i: (0, 0)),
            pl.BlockSpec((1,), lambda i: (0,)),
        ],
        out_specs=pl.BlockSpec((BLOCK, 1), lambda i: (i, 0)),
        out_shape=jax.ShapeDtypeStruct((n, 1), x.dtype),
        scratch_shapes=[pltpu.VMEM((F_IN, 2 * FILTERS), jnp.float32)],
        compiler_params=pltpu.CompilerParams(
            dimension_semantics=("arbitrary",),
        ),
    )(x, W_z, W_h, b_z, b_h, W_lin, b_lin)
    return out
